# speedup vs baseline: 1.9674x; 1.9674x over previous
import jax
import jax.numpy as jnp
from jax.experimental import pallas as pl
from jax.experimental.pallas import tpu as pltpu

_SEQ_BLOCK = 512


def _body(x_ref, emb_ref, o_ref):
    o_ref[...] = x_ref[...] + emb_ref[...][None, :, :]


def kernel(x, pos_emb):
    batch, seq_len, d = x.shape
    grid = (seq_len // _SEQ_BLOCK,)
    return pl.pallas_call(
        _body,
        grid=grid,
        in_specs=[
            pl.BlockSpec((batch, _SEQ_BLOCK, d), lambda s: (0, s, 0)),
            pl.BlockSpec((_SEQ_BLOCK, d), lambda s: (s, 0)),
        ],
        out_specs=pl.BlockSpec((batch, _SEQ_BLOCK, d), lambda s: (0, s, 0)),
        out_shape=jax.ShapeDtypeStruct(x.shape, x.dtype),
    )(x, pos_emb)
